# baseline (speedup 1.0000x reference)
import jax
import jax.numpy as jnp
from jax import lax
from jax.experimental import pallas as pl
from jax.experimental.pallas import tpu as pltpu
from jax.experimental.pallas import tpu_sc as plsc

NN = 10000
EE = 320000
ET = EE + NN
H = 512
Q = 128
HF = 256
K = 128
TILES = 16
WPT = 162
M16 = WPT * K
EP = TILES * M16
NP = 10240
GRP = K // 16
KB = 128
WPTB = M16 // KB
GRPB = KB // 16

_mesh = plsc.VectorSubcoreMesh(core_axis_name="c", subcore_axis_name="s")
_sc_params = pltpu.CompilerParams(needs_layout_passes=False)


def _split_outs(xl, xr, outs):
    xlb = xl.astype(jnp.bfloat16)
    xrb = xr.astype(jnp.bfloat16)
    outs[0][...] = jnp.stack([xlb[:, :HF], xlb[:, HF:]], axis=0)
    outs[1][...] = jnp.stack([xrb[:, :HF], xrb[:, HF:]], axis=0)
    outs[2][...] = jnp.stack(
        [xl[:, q * Q:(q + 1) * Q] for q in range(4)], axis=0)


def _head_body(nodes_ref, wpre_ref, wlin_ref, wl_ref, wr_ref, *outs):
    x = jnp.maximum(jnp.dot(nodes_ref[...], wpre_ref[...],
                            preferred_element_type=jnp.float32), 0.0)
    for _ in range(3):
        x = jnp.maximum(jnp.dot(x, wlin_ref[...],
                                preferred_element_type=jnp.float32), 0.0)
    xl = jnp.dot(x, wl_ref[...], preferred_element_type=jnp.float32)
    xr = jnp.dot(x, wr_ref[...], preferred_element_type=jnp.float32)
    _split_outs(xl, xr, outs)


def _xspecs(bm):
    return [
        pl.BlockSpec((2, bm, HF), lambda i: (0, i, 0)),
        pl.BlockSpec((2, bm, HF), lambda i: (0, i, 0)),
        pl.BlockSpec((4, bm, Q), lambda i: (0, i, 0)),
    ]


_XSHAPES = [
    jax.ShapeDtypeStruct((2, NN, HF), jnp.bfloat16),
    jax.ShapeDtypeStruct((2, NN, HF), jnp.bfloat16),
    jax.ShapeDtypeStruct((4, NN, Q), jnp.float32),
]


def _head(nodes, Wpre, Wlin, Wl0, Wr0):
    bm = 2000
    return pl.pallas_call(
        _head_body,
        grid=(NN // bm,),
        in_specs=[
            pl.BlockSpec((bm, 128), lambda i: (i, 0)),
            pl.BlockSpec((128, H), lambda i: (0, 0)),
            pl.BlockSpec((H, H), lambda i: (0, 0)),
            pl.BlockSpec((H, H), lambda i: (0, 0)),
            pl.BlockSpec((H, H), lambda i: (0, 0)),
        ],
        out_specs=_xspecs(bm),
        out_shape=_XSHAPES,
    )(nodes, Wpre, Wlin, Wl0, Wr0)


def _combine_body(n0, n1, n2, n3, den_ref, b_ref, wl_ref, wr_ref, *outs):
    num = jnp.concatenate([n0[...], n1[...], n2[...], n3[...]], axis=1)
    x = num / den_ref[...] + b_ref[...]
    xl = jnp.dot(x, wl_ref[...], preferred_element_type=jnp.float32)
    xr = jnp.dot(x, wr_ref[...], preferred_element_type=jnp.float32)
    _split_outs(xl, xr, outs)


def _combine(nq, den2, bl, Wln, Wrn):
    bm = 2000
    qspec = pl.BlockSpec((bm, Q), lambda i: (i, 0))
    return pl.pallas_call(
        _combine_body,
        grid=(NN // bm,),
        in_specs=[qspec] * 4 + [
            pl.BlockSpec((bm, 1), lambda i: (i, 0)),
            pl.BlockSpec((1, H), lambda i: (0, 0)),
            pl.BlockSpec((H, H), lambda i: (0, 0)),
            pl.BlockSpec((H, H), lambda i: (0, 0)),
        ],
        out_specs=_xspecs(bm),
        out_shape=_XSHAPES,
    )(*nq, den2, bl, Wln, Wrn)


def _final_body(n0, n1, n2, n3, den_ref, b_ref, wc_ref, bc_ref, x_out, o_out):
    num = jnp.concatenate([n0[...], n1[...], n2[...], n3[...]], axis=1)
    x = num / den_ref[...] + b_ref[...]
    x_out[...] = x
    o_out[...] = jnp.dot(x, wc_ref[...],
                         preferred_element_type=jnp.float32) + bc_ref[...]


def _final(nq, den2, bl, Wc, bc2):
    bm = 2000
    qspec = pl.BlockSpec((bm, Q), lambda i: (i, 0))
    nout = Wc.shape[1]
    return pl.pallas_call(
        _final_body,
        grid=(NN // bm,),
        in_specs=[qspec] * 4 + [
            pl.BlockSpec((bm, 1), lambda i: (i, 0)),
            pl.BlockSpec((1, H), lambda i: (0, 0)),
            pl.BlockSpec((H, nout), lambda i: (0, 0)),
            pl.BlockSpec((1, nout), lambda i: (0, 0)),
        ],
        out_specs=[
            pl.BlockSpec((bm, H), lambda i: (i, 0)),
            pl.BlockSpec((bm, nout), lambda i: (i, 0)),
        ],
        out_shape=[
            jax.ShapeDtypeStruct((NN, H), jnp.float32),
            jax.ShapeDtypeStruct((NN, nout), jnp.float32),
        ],
    )(*nq, den2, bl, Wc, bc2)


def _score_body(xlh, xrh, att_hbm, src_hbm, dst_hbm, p_hbm,
                is0, is1, id0, id1, rl0, rl1, rr0, rr1, score_v, att_v, pbuf,
                semi0, semi1, semg0, semg1):
    cid = lax.axis_index("c")
    sid = lax.axis_index("s")
    pltpu.sync_copy(att_hbm, att_v)
    lanes = lax.iota(jnp.int32, 16)
    lanes16 = lanes * 16
    att_off = cid * HF
    p_off = cid * EP
    tile0 = sid * M16
    roff = jnp.full((16,), cid * NN, jnp.int32)

    bufs = ((is0, id0, rl0, rr0, semi0, semg0),
            (is1, id1, rl1, rr1, semi1, semg1))

    def wbase(w):
        return tile0 + jnp.minimum(w, WPT - 1) * K

    def issue_idx(w, b):
        is_b, id_b, _, _, semi, _ = bufs[b]
        base = wbase(w)
        pltpu.async_copy(src_hbm.at[pl.ds(base, K)], is_b, semi)
        pltpu.async_copy(dst_hbm.at[pl.ds(base, K)], id_b, semi)

    def wait_idx_bump(b):
        is_b, id_b, _, _, semi, _ = bufs[b]
        pltpu.make_async_copy(src_hbm.at[pl.ds(0, K)], is_b, semi).wait()
        pltpu.make_async_copy(dst_hbm.at[pl.ds(0, K)], id_b, semi).wait()
        for g in range(GRP):
            sl = pl.ds(g * 16, 16)
            is_b[sl] = is_b[sl] + roff
            id_b[sl] = id_b[sl] + roff

    def issue_gath(b):
        is_b, id_b, rl, rr, _, semg = bufs[b]
        pltpu.async_copy(xlh.at[is_b], rl, semg)
        pltpu.async_copy(xrh.at[id_b], rr, semg)

    def wait_gath(b):
        is_b, id_b, rl, rr, _, semg = bufs[b]
        pltpu.make_async_copy(xlh.at[is_b], rl, semg).wait()
        pltpu.make_async_copy(xrh.at[id_b], rr, semg).wait()

    ae = [att_v[pl.ds(att_off + j * 32, 16)] for j in range(8)]
    ao = [att_v[pl.ds(att_off + j * 32 + 16, 16)] for j in range(8)]

    def compute(w, b):
        _, _, rl, rr, _, _ = bufs[b]
        base = wbase(w)

        def grp(g, carry):
            for e16 in range(16):
                e = g * 16 + e16
                acc = jnp.zeros((16,), jnp.float32)
                for j in range(8):
                    la = plsc.bitcast(rl[e, pl.ds(j * 16, 16)], jnp.bfloat16)
                    ra = plsc.bitcast(rr[e, pl.ds(j * 16, 16)], jnp.bfloat16)
                    m = la + ra
                    lk = jnp.maximum(m, jnp.bfloat16(0.2) * m)
                    l0, l1 = plsc.unpack(lk, format=plsc.PackFormat.INTERLEAVED)
                    acc = acc + l0 * ae[j] + l1 * ao[j]
                pbuf[pl.ds(e16 * 16, 16)] = acc
            tot = jnp.zeros((16,), jnp.float32)
            for l2 in range(16):
                tot = tot + plsc.load_gather(pbuf, [lanes16 + l2])
            gid = base + g * 16 + lanes
            tot = jnp.where(gid < ET, tot, -1e30)
            score_v[pl.ds(g * 16, 16)] = tot
            return carry

        lax.fori_loop(0, GRP, grp, 0)
        pltpu.sync_copy(score_v, p_hbm.at[pl.ds(p_off + base, K)])

    issue_idx(0, 0)
    wait_idx_bump(0)
    issue_gath(0)
    issue_idx(1, 1)

    def outer(w2, carry):
        for b in range(2):
            w = w2 * 2 + b
            wait_gath(b)
            wait_idx_bump(1 - b)
            issue_gath(1 - b)
            issue_idx(w + 2, b)
            compute(w, b)
        return carry

    lax.fori_loop(0, WPT // 2, outer, 0)
    wait_gath(0)
    wait_idx_bump(1)


def _score(xlh, xrh, att_l, src, dst):
    f = pl.kernel(
        _score_body,
        out_type=jax.ShapeDtypeStruct((2 * EP,), jnp.float32),
        mesh=_mesh,
        compiler_params=_sc_params,
        scratch_types=[
            pltpu.VMEM((K,), jnp.int32),
            pltpu.VMEM((K,), jnp.int32),
            pltpu.VMEM((K,), jnp.int32),
            pltpu.VMEM((K,), jnp.int32),
            pltpu.VMEM((K, HF // 2), jnp.int32),
            pltpu.VMEM((K, HF // 2), jnp.int32),
            pltpu.VMEM((K, HF // 2), jnp.int32),
            pltpu.VMEM((K, HF // 2), jnp.int32),
            pltpu.VMEM((K,), jnp.float32),
            pltpu.VMEM((H,), jnp.float32),
            pltpu.VMEM((256,), jnp.float32),
            pltpu.SemaphoreType.DMA,
            pltpu.SemaphoreType.DMA,
            pltpu.SemaphoreType.DMA,
            pltpu.SemaphoreType.DMA,
        ],
    )
    return f(xlh, xrh, att_l, src, dst)


def _agg_body(xq_hbm, src_hbm, dst_hbm, p_hbm, num_hbm, den_hbm,
              is0, is1, id0, id1, p00, p01, p10, p11, rw0, rw1,
              wb0, wb1, zbuf, zden, sh_num, sh_den,
              semi0, semi1, semg0, semg1, sems0, sems1):
    cid = lax.axis_index("c")
    sid = lax.axis_index("s")
    tile0 = sid * M16

    bufs = ((is0, id0, p00, p10, rw0, wb0, semi0, semg0, sems0),
            (is1, id1, p01, p11, rw1, wb1, semi1, semg1, sems1))

    def zr(r, carry):
        for j in range(8):
            zbuf[r, pl.ds(j * 16, 16)] = jnp.zeros((16,), jnp.float32)
        return carry

    lax.fori_loop(0, 64, zr, 0)

    def zd(g, carry):
        zden[pl.ds(g * 16, 16)] = jnp.zeros((16,), jnp.float32)
        return carry

    lax.fori_loop(0, 40, zd, 0)

    def wbase(w):
        return tile0 + jnp.minimum(w, WPTB - 1) * KB

    def issue_idx(w, b):
        is_b, _, p0, p1, _, _, semi, _, _ = bufs[b]
        base = wbase(w)
        pltpu.async_copy(src_hbm.at[pl.ds(base, KB)], is_b, semi)
        pltpu.async_copy(p_hbm.at[pl.ds(base, KB)], p0, semi)
        pltpu.async_copy(p_hbm.at[pl.ds(EP + base, KB)], p1, semi)

    def wait_idx_bump(b, roff):
        is_b, _, p0, p1, _, _, semi, _, _ = bufs[b]
        pltpu.make_async_copy(src_hbm.at[pl.ds(0, KB)], is_b, semi).wait()
        pltpu.make_async_copy(p_hbm.at[pl.ds(0, KB)], p0, semi).wait()
        pltpu.make_async_copy(p_hbm.at[pl.ds(0, KB)], p1, semi).wait()
        for g in range(GRPB):
            sl = pl.ds(g * 16, 16)
            is_b[sl] = is_b[sl] + roff

    def issue_gath(w, b):
        is_b, id_b, _, _, rw, _, _, semg, _ = bufs[b]
        base = wbase(w)
        pltpu.async_copy(xq_hbm.at[is_b], rw, semg)
        pltpu.async_copy(dst_hbm.at[pl.ds(base, KB)], id_b, semg)

    def wait_gath(b):
        is_b, id_b, _, _, rw, _, _, semg, _ = bufs[b]
        pltpu.make_async_copy(xq_hbm.at[is_b], rw, semg).wait()
        pltpu.make_async_copy(dst_hbm.at[pl.ds(0, KB)], id_b, semg).wait()

    def issue_scat(b, do_den):
        _, id_b, _, _, rw, wb, _, _, sems = bufs[b]
        pltpu.async_copy(rw, sh_num.at[id_b], sems, add=True)
        if do_den:
            @pl.when(cid == 0)
            def _():
                pltpu.async_copy(wb, sh_den.at[id_b], sems, add=True)

    def wait_scat(b, do_den):
        _, id_b, _, _, rw, wb, _, _, sems = bufs[b]
        pltpu.make_async_copy(rw, sh_num.at[id_b], sems).wait()
        if do_den:
            @pl.when(cid == 0)
            def _():
                pltpu.make_async_copy(wb, sh_den.at[id_b], sems).wait()

    def compute(b):
        _, _, p0, p1, rw, wb, _, _, _ = bufs[b]

        def grp(g, carry):
            wv = jnp.exp(p0[pl.ds(g * 16, 16)] + p1[pl.ds(g * 16, 16)])
            wb[pl.ds(g * 16, 16)] = wv
            for e16 in range(16):
                e = g * 16 + e16
                w_e = plsc.load_gather(wb, [jnp.full((16,), e, jnp.int32)])
                for j in range(8):
                    rw[e, pl.ds(j * 16, 16)] = rw[e, pl.ds(j * 16, 16)] * w_e
            return carry

        lax.fori_loop(0, GRPB, grp, 0)

    def sub_pass(cc):
        do_den = cc == 0
        roff = jnp.full((16,), (2 * cid + cc) * NN, jnp.int32)
        dump_off = (2 * cid + cc) * NP + sid * 640
        for t in range(10):
            pltpu.sync_copy(zbuf, sh_num.at[pl.ds(sid * 640 + t * 64, 64)])
        if do_den:
            @pl.when(cid == 0)
            def _():
                pltpu.sync_copy(zden, sh_den.at[pl.ds(sid * 640, 640)])
        _, idp, _, _, rwp, wbp, _, _, _ = bufs[1]

        def zrow(r, carry):
            for j in range(8):
                rwp[r, pl.ds(j * 16, 16)] = jnp.zeros((16,), jnp.float32)
            return carry

        lax.fori_loop(0, KB, zrow, 0)

        def zsml(g, carry):
            wbp[pl.ds(g * 16, 16)] = jnp.zeros((16,), jnp.float32)
            idp[pl.ds(g * 16, 16)] = jnp.zeros((16,), jnp.int32)
            return carry

        lax.fori_loop(0, GRPB, zsml, 0)
        plsc.subcore_barrier()
        issue_scat(1, do_den)

        issue_idx(0, 0)
        wait_idx_bump(0, roff)
        issue_gath(0, 0)
        issue_idx(1, 1)

        def outer(w2, carry):
            for b in range(2):
                w = w2 * 2 + b
                wait_gath(b)
                wait_idx_bump(1 - b, roff)
                wait_scat(1 - b, do_den)
                issue_gath(w + 1, 1 - b)
                compute(b)
                issue_scat(b, do_den)
                issue_idx(w + 2, b)
            return carry

        lax.fori_loop(0, WPTB // 2, outer, 0)
        wait_gath(0)
        wait_idx_bump(1, roff)
        wait_scat(1, do_den)
        plsc.subcore_barrier()
        pltpu.sync_copy(sh_num.at[pl.ds(sid * 640, 640)],
                        num_hbm.at[pl.ds(dump_off, 640)])
        if do_den:
            @pl.when(cid == 0)
            def _():
                pltpu.sync_copy(sh_den.at[pl.ds(sid * 640, 640)],
                                den_hbm.at[pl.ds(sid * 640, 640)])
        plsc.subcore_barrier()

    sub_pass(0)
    sub_pass(1)


def _agg(xq, src, dst, p):
    f = pl.kernel(
        _agg_body,
        out_type=[jax.ShapeDtypeStruct((4 * NP, Q), jnp.float32),
                  jax.ShapeDtypeStruct((NP,), jnp.float32)],
        mesh=_mesh,
        compiler_params=_sc_params,
        scratch_types=[
            pltpu.VMEM((KB,), jnp.int32),
            pltpu.VMEM((KB,), jnp.int32),
            pltpu.VMEM((KB,), jnp.int32),
            pltpu.VMEM((KB,), jnp.int32),
            pltpu.VMEM((KB,), jnp.float32),
            pltpu.VMEM((KB,), jnp.float32),
            pltpu.VMEM((KB,), jnp.float32),
            pltpu.VMEM((KB,), jnp.float32),
            pltpu.VMEM((KB, Q), jnp.float32),
            pltpu.VMEM((KB, Q), jnp.float32),
            pltpu.VMEM((KB,), jnp.float32),
            pltpu.VMEM((KB,), jnp.float32),
            pltpu.VMEM((64, Q), jnp.float32),
            pltpu.VMEM((640,), jnp.float32),
            pltpu.VMEM_SHARED((NP, Q), jnp.float32),
            pltpu.VMEM_SHARED((NP,), jnp.float32),
            pltpu.SemaphoreType.DMA,
            pltpu.SemaphoreType.DMA,
            pltpu.SemaphoreType.DMA,
            pltpu.SemaphoreType.DMA,
            pltpu.SemaphoreType.DMA,
            pltpu.SemaphoreType.DMA,
        ],
    )
    return f(xq, src, dst, p)


def _att_perm():
    idx = []
    for blk in range(H // 32):
        idx.extend(blk * 32 + 2 * k for k in range(16))
        idx.extend(blk * 32 + 2 * k + 1 for k in range(16))
    return jnp.array(idx, jnp.int32)


def kernel(nodes, edge_index, Wpre, Wlin, Wl, Wr, att, b, Wc, bc):
    att_r = att[:, _att_perm()]
    loop = jnp.arange(NN, dtype=edge_index.dtype)
    pad = jnp.zeros((EP - ET,), dtype=edge_index.dtype)
    src = jnp.concatenate([edge_index[0], loop, pad])
    dst = jnp.concatenate([edge_index[1], loop, pad])

    xlh3, xrh3, xlq3 = _head(nodes, Wpre, Wlin, Wl[0], Wr[0])
    x = out = None
    for l in range(4):
        xlh = lax.bitcast_convert_type(
            xlh3.reshape(2 * NN, HF // 2, 2), jnp.int32)
        xrh = lax.bitcast_convert_type(
            xrh3.reshape(2 * NN, HF // 2, 2), jnp.int32)
        xlq = xlq3.reshape(4 * NN, Q)
        p = _score(xlh, xrh, att_r[l], src, dst)
        num, den = _agg(xlq, src, dst, p)
        num4 = num.reshape(4, NP, Q)
        nq = tuple(num4[q] for q in range(4))
        den2 = den.reshape(NP, 1)
        bl = b[l].reshape(1, H)
        if l < 3:
            xlh3, xrh3, xlq3 = _combine(nq, den2, bl, Wl[l + 1], Wr[l + 1])
        else:
            x, out = _final(nq, den2, bl, Wc, bc.reshape(1, Wc.shape[1]))
    return (x, out)


# speedup vs baseline: 1.1672x; 1.1672x over previous
import jax
import jax.numpy as jnp
from jax import lax
from jax.experimental import pallas as pl
from jax.experimental.pallas import tpu as pltpu
from jax.experimental.pallas import tpu_sc as plsc

NN = 10000
EE = 320000
ET = EE + NN
H = 512
Q = 128
HF = 256
K = 96
TILES = 16
WPT = 216
M16 = WPT * K
EP = TILES * M16
NP = 10240
GRP = K // 16
KB = 128
WPTB = M16 // KB
GRPB = KB // 16

_mesh = plsc.VectorSubcoreMesh(core_axis_name="c", subcore_axis_name="s")
_sc_params = pltpu.CompilerParams(needs_layout_passes=False)


def _split_outs(xl, xr, outs):
    outs[0][...] = jnp.stack([xl[:, :HF], xl[:, HF:]], axis=0)
    outs[1][...] = jnp.stack([xr[:, :HF], xr[:, HF:]], axis=0)
    outs[2][...] = jnp.stack(
        [xl[:, q * Q:(q + 1) * Q] for q in range(4)], axis=0)


def _head_body(nodes_ref, wpre_ref, wlin_ref, wl_ref, wr_ref, *outs):
    x = jnp.maximum(jnp.dot(nodes_ref[...], wpre_ref[...],
                            preferred_element_type=jnp.float32), 0.0)
    for _ in range(3):
        x = jnp.maximum(jnp.dot(x, wlin_ref[...],
                                preferred_element_type=jnp.float32), 0.0)
    xl = jnp.dot(x, wl_ref[...], preferred_element_type=jnp.float32)
    xr = jnp.dot(x, wr_ref[...], preferred_element_type=jnp.float32)
    _split_outs(xl, xr, outs)


def _xspecs(bm):
    return [
        pl.BlockSpec((2, bm, HF), lambda i: (0, i, 0)),
        pl.BlockSpec((2, bm, HF), lambda i: (0, i, 0)),
        pl.BlockSpec((4, bm, Q), lambda i: (0, i, 0)),
    ]


_XSHAPES = [
    jax.ShapeDtypeStruct((2, NN, HF), jnp.float32),
    jax.ShapeDtypeStruct((2, NN, HF), jnp.float32),
    jax.ShapeDtypeStruct((4, NN, Q), jnp.float32),
]


def _head(nodes, Wpre, Wlin, Wl0, Wr0):
    bm = 2000
    return pl.pallas_call(
        _head_body,
        grid=(NN // bm,),
        in_specs=[
            pl.BlockSpec((bm, 128), lambda i: (i, 0)),
            pl.BlockSpec((128, H), lambda i: (0, 0)),
            pl.BlockSpec((H, H), lambda i: (0, 0)),
            pl.BlockSpec((H, H), lambda i: (0, 0)),
            pl.BlockSpec((H, H), lambda i: (0, 0)),
        ],
        out_specs=_xspecs(bm),
        out_shape=_XSHAPES,
    )(nodes, Wpre, Wlin, Wl0, Wr0)


def _combine_body(n0, n1, n2, n3, den_ref, b_ref, wl_ref, wr_ref, *outs):
    num = jnp.concatenate([n0[...], n1[...], n2[...], n3[...]], axis=1)
    x = num / den_ref[...] + b_ref[...]
    xl = jnp.dot(x, wl_ref[...], preferred_element_type=jnp.float32)
    xr = jnp.dot(x, wr_ref[...], preferred_element_type=jnp.float32)
    _split_outs(xl, xr, outs)


def _combine(nq, den2, bl, Wln, Wrn):
    bm = 2000
    qspec = pl.BlockSpec((bm, Q), lambda i: (i, 0))
    return pl.pallas_call(
        _combine_body,
        grid=(NN // bm,),
        in_specs=[qspec] * 4 + [
            pl.BlockSpec((bm, 1), lambda i: (i, 0)),
            pl.BlockSpec((1, H), lambda i: (0, 0)),
            pl.BlockSpec((H, H), lambda i: (0, 0)),
            pl.BlockSpec((H, H), lambda i: (0, 0)),
        ],
        out_specs=_xspecs(bm),
        out_shape=_XSHAPES,
    )(*nq, den2, bl, Wln, Wrn)


def _final_body(n0, n1, n2, n3, den_ref, b_ref, wc_ref, bc_ref, x_out, o_out):
    num = jnp.concatenate([n0[...], n1[...], n2[...], n3[...]], axis=1)
    x = num / den_ref[...] + b_ref[...]
    x_out[...] = x
    o_out[...] = jnp.dot(x, wc_ref[...],
                         preferred_element_type=jnp.float32) + bc_ref[...]


def _final(nq, den2, bl, Wc, bc2):
    bm = 2000
    qspec = pl.BlockSpec((bm, Q), lambda i: (i, 0))
    nout = Wc.shape[1]
    return pl.pallas_call(
        _final_body,
        grid=(NN // bm,),
        in_specs=[qspec] * 4 + [
            pl.BlockSpec((bm, 1), lambda i: (i, 0)),
            pl.BlockSpec((1, H), lambda i: (0, 0)),
            pl.BlockSpec((H, nout), lambda i: (0, 0)),
            pl.BlockSpec((1, nout), lambda i: (0, 0)),
        ],
        out_specs=[
            pl.BlockSpec((bm, H), lambda i: (i, 0)),
            pl.BlockSpec((bm, nout), lambda i: (i, 0)),
        ],
        out_shape=[
            jax.ShapeDtypeStruct((NN, H), jnp.float32),
            jax.ShapeDtypeStruct((NN, nout), jnp.float32),
        ],
    )(*nq, den2, bl, Wc, bc2)


def _score_body(xlh, xrh, att_hbm, src_hbm, dst_hbm, p_hbm,
                is0, is1, id0, id1, rl0, rl1, rr0, rr1, score_v, att_v, pbuf,
                semi0, semi1, semg0, semg1):
    cid = lax.axis_index("c")
    sid = lax.axis_index("s")
    pltpu.sync_copy(att_hbm, att_v)
    lanes = lax.iota(jnp.int32, 16)
    lanes16 = lanes * 16
    att_off = cid * HF
    p_off = cid * EP
    tile0 = sid * M16
    roff = jnp.full((16,), cid * NN, jnp.int32)

    bufs = ((is0, id0, rl0, rr0, semi0, semg0),
            (is1, id1, rl1, rr1, semi1, semg1))

    def wbase(w):
        return tile0 + jnp.minimum(w, WPT - 1) * K

    def issue_idx(w, b):
        is_b, id_b, _, _, semi, _ = bufs[b]
        base = wbase(w)
        pltpu.async_copy(src_hbm.at[pl.ds(base, K)], is_b, semi)
        pltpu.async_copy(dst_hbm.at[pl.ds(base, K)], id_b, semi)

    def wait_idx_bump(b):
        is_b, id_b, _, _, semi, _ = bufs[b]
        pltpu.make_async_copy(src_hbm.at[pl.ds(0, K)], is_b, semi).wait()
        pltpu.make_async_copy(dst_hbm.at[pl.ds(0, K)], id_b, semi).wait()
        for g in range(GRP):
            sl = pl.ds(g * 16, 16)
            is_b[sl] = is_b[sl] + roff
            id_b[sl] = id_b[sl] + roff

    def issue_gath(b):
        is_b, id_b, rl, rr, _, semg = bufs[b]
        pltpu.async_copy(xlh.at[is_b], rl, semg)
        pltpu.async_copy(xrh.at[id_b], rr, semg)

    def wait_gath(b):
        is_b, id_b, rl, rr, _, semg = bufs[b]
        pltpu.make_async_copy(xlh.at[is_b], rl, semg).wait()
        pltpu.make_async_copy(xrh.at[id_b], rr, semg).wait()

    areg = [att_v[pl.ds(att_off + j * 16, 16)] for j in range(16)]

    def compute(w, b):
        _, _, rl, rr, _, _ = bufs[b]
        base = wbase(w)

        def grp(g, carry):
            for e16 in range(16):
                e = g * 16 + e16
                acc = jnp.zeros((16,), jnp.float32)
                for j in range(16):
                    m = rl[e, pl.ds(j * 16, 16)] + rr[e, pl.ds(j * 16, 16)]
                    acc = acc + jnp.maximum(m, 0.2 * m) * areg[j]
                pbuf[pl.ds(e16 * 16, 16)] = acc
            tot = jnp.zeros((16,), jnp.float32)
            for l2 in range(16):
                tot = tot + plsc.load_gather(pbuf, [lanes16 + l2])
            gid = base + g * 16 + lanes
            tot = jnp.where(gid < ET, tot, -1e30)
            score_v[pl.ds(g * 16, 16)] = tot
            return carry

        lax.fori_loop(0, GRP, grp, 0)
        pltpu.sync_copy(score_v, p_hbm.at[pl.ds(p_off + base, K)])

    issue_idx(0, 0)
    wait_idx_bump(0)
    issue_gath(0)
    issue_idx(1, 1)

    def outer(w2, carry):
        for b in range(2):
            w = w2 * 2 + b
            wait_gath(b)
            wait_idx_bump(1 - b)
            issue_gath(1 - b)
            issue_idx(w + 2, b)
            compute(w, b)
        return carry

    lax.fori_loop(0, WPT // 2, outer, 0)
    wait_gath(0)
    wait_idx_bump(1)


def _score(xlh, xrh, att_l, src, dst):
    f = pl.kernel(
        _score_body,
        out_type=jax.ShapeDtypeStruct((2 * EP,), jnp.float32),
        mesh=_mesh,
        compiler_params=_sc_params,
        scratch_types=[
            pltpu.VMEM((K,), jnp.int32),
            pltpu.VMEM((K,), jnp.int32),
            pltpu.VMEM((K,), jnp.int32),
            pltpu.VMEM((K,), jnp.int32),
            pltpu.VMEM((K, HF), jnp.float32),
            pltpu.VMEM((K, HF), jnp.float32),
            pltpu.VMEM((K, HF), jnp.float32),
            pltpu.VMEM((K, HF), jnp.float32),
            pltpu.VMEM((K,), jnp.float32),
            pltpu.VMEM((H,), jnp.float32),
            pltpu.VMEM((256,), jnp.float32),
            pltpu.SemaphoreType.DMA,
            pltpu.SemaphoreType.DMA,
            pltpu.SemaphoreType.DMA,
            pltpu.SemaphoreType.DMA,
        ],
    )
    return f(xlh, xrh, att_l, src, dst)


def _agg_body(xq_hbm, src_hbm, dst_hbm, p_hbm, num_hbm, den_hbm,
              is0, is1, id0, id1, p00, p01, p10, p11, rw0, rw1,
              wb0, wb1, zbuf, zden, sh_num, sh_den,
              semi0, semi1, semg0, semg1, sems0, sems1):
    cid = lax.axis_index("c")
    sid = lax.axis_index("s")
    tile0 = sid * M16

    bufs = ((is0, id0, p00, p10, rw0, wb0, semi0, semg0, sems0),
            (is1, id1, p01, p11, rw1, wb1, semi1, semg1, sems1))

    def zr(r, carry):
        for j in range(8):
            zbuf[r, pl.ds(j * 16, 16)] = jnp.zeros((16,), jnp.float32)
        return carry

    lax.fori_loop(0, 64, zr, 0)

    def zd(g, carry):
        zden[pl.ds(g * 16, 16)] = jnp.zeros((16,), jnp.float32)
        return carry

    lax.fori_loop(0, 40, zd, 0)

    def wbase(w):
        return tile0 + jnp.minimum(w, WPTB - 1) * KB

    def issue_idx(w, b):
        is_b, _, p0, p1, _, _, semi, _, _ = bufs[b]
        base = wbase(w)
        pltpu.async_copy(src_hbm.at[pl.ds(base, KB)], is_b, semi)
        pltpu.async_copy(p_hbm.at[pl.ds(base, KB)], p0, semi)
        pltpu.async_copy(p_hbm.at[pl.ds(EP + base, KB)], p1, semi)

    def wait_idx_bump(b, roff):
        is_b, _, p0, p1, _, _, semi, _, _ = bufs[b]
        pltpu.make_async_copy(src_hbm.at[pl.ds(0, KB)], is_b, semi).wait()
        pltpu.make_async_copy(p_hbm.at[pl.ds(0, KB)], p0, semi).wait()
        pltpu.make_async_copy(p_hbm.at[pl.ds(0, KB)], p1, semi).wait()
        for g in range(GRPB):
            sl = pl.ds(g * 16, 16)
            is_b[sl] = is_b[sl] + roff

    def issue_gath(w, b):
        is_b, id_b, _, _, rw, _, _, semg, _ = bufs[b]
        base = wbase(w)
        pltpu.async_copy(xq_hbm.at[is_b], rw, semg)
        pltpu.async_copy(dst_hbm.at[pl.ds(base, KB)], id_b, semg)

    def wait_gath(b):
        is_b, id_b, _, _, rw, _, _, semg, _ = bufs[b]
        pltpu.make_async_copy(xq_hbm.at[is_b], rw, semg).wait()
        pltpu.make_async_copy(dst_hbm.at[pl.ds(0, KB)], id_b, semg).wait()

    def issue_scat(b, do_den):
        _, id_b, _, _, rw, wb, _, _, sems = bufs[b]
        pltpu.async_copy(rw, sh_num.at[id_b], sems, add=True)
        if do_den:
            @pl.when(cid == 0)
            def _():
                pltpu.async_copy(wb, sh_den.at[id_b], sems, add=True)

    def wait_scat(b, do_den):
        _, id_b, _, _, rw, wb, _, _, sems = bufs[b]
        pltpu.make_async_copy(rw, sh_num.at[id_b], sems).wait()
        if do_den:
            @pl.when(cid == 0)
            def _():
                pltpu.make_async_copy(wb, sh_den.at[id_b], sems).wait()

    def compute(b):
        _, _, p0, p1, rw, wb, _, _, _ = bufs[b]

        def grp(g, carry):
            wv = jnp.exp(p0[pl.ds(g * 16, 16)] + p1[pl.ds(g * 16, 16)])
            wb[pl.ds(g * 16, 16)] = wv
            for e16 in range(16):
                e = g * 16 + e16
                w_e = plsc.load_gather(wb, [jnp.full((16,), e, jnp.int32)])
                for j in range(8):
                    rw[e, pl.ds(j * 16, 16)] = rw[e, pl.ds(j * 16, 16)] * w_e
            return carry

        lax.fori_loop(0, GRPB, grp, 0)

    def sub_pass(cc):
        do_den = cc == 0
        roff = jnp.full((16,), (2 * cid + cc) * NN, jnp.int32)
        dump_off = (2 * cid + cc) * NP + sid * 640
        for t in range(10):
            pltpu.sync_copy(zbuf, sh_num.at[pl.ds(sid * 640 + t * 64, 64)])
        if do_den:
            @pl.when(cid == 0)
            def _():
                pltpu.sync_copy(zden, sh_den.at[pl.ds(sid * 640, 640)])
        _, idp, _, _, rwp, wbp, _, _, _ = bufs[1]

        def zrow(r, carry):
            for j in range(8):
                rwp[r, pl.ds(j * 16, 16)] = jnp.zeros((16,), jnp.float32)
            return carry

        lax.fori_loop(0, KB, zrow, 0)

        def zsml(g, carry):
            wbp[pl.ds(g * 16, 16)] = jnp.zeros((16,), jnp.float32)
            idp[pl.ds(g * 16, 16)] = jnp.zeros((16,), jnp.int32)
            return carry

        lax.fori_loop(0, GRPB, zsml, 0)
        plsc.subcore_barrier()
        issue_scat(1, do_den)

        issue_idx(0, 0)
        wait_idx_bump(0, roff)
        issue_gath(0, 0)
        issue_idx(1, 1)

        def outer(w2, carry):
            for b in range(2):
                w = w2 * 2 + b
                wait_gath(b)
                wait_idx_bump(1 - b, roff)
                wait_scat(1 - b, do_den)
                issue_gath(w + 1, 1 - b)
                compute(b)
                issue_scat(b, do_den)
                issue_idx(w + 2, b)
            return carry

        lax.fori_loop(0, WPTB // 2, outer, 0)
        wait_gath(0)
        wait_idx_bump(1, roff)
        wait_scat(1, do_den)
        plsc.subcore_barrier()
        pltpu.sync_copy(sh_num.at[pl.ds(sid * 640, 640)],
                        num_hbm.at[pl.ds(dump_off, 640)])
        if do_den:
            @pl.when(cid == 0)
            def _():
                pltpu.sync_copy(sh_den.at[pl.ds(sid * 640, 640)],
                                den_hbm.at[pl.ds(sid * 640, 640)])
        plsc.subcore_barrier()

    sub_pass(0)
    sub_pass(1)


def _agg(xq, src, dst, p):
    f = pl.kernel(
        _agg_body,
        out_type=[jax.ShapeDtypeStruct((4 * NP, Q), jnp.float32),
                  jax.ShapeDtypeStruct((NP,), jnp.float32)],
        mesh=_mesh,
        compiler_params=_sc_params,
        scratch_types=[
            pltpu.VMEM((KB,), jnp.int32),
            pltpu.VMEM((KB,), jnp.int32),
            pltpu.VMEM((KB,), jnp.int32),
            pltpu.VMEM((KB,), jnp.int32),
            pltpu.VMEM((KB,), jnp.float32),
            pltpu.VMEM((KB,), jnp.float32),
            pltpu.VMEM((KB,), jnp.float32),
            pltpu.VMEM((KB,), jnp.float32),
            pltpu.VMEM((KB, Q), jnp.float32),
            pltpu.VMEM((KB, Q), jnp.float32),
            pltpu.VMEM((KB,), jnp.float32),
            pltpu.VMEM((KB,), jnp.float32),
            pltpu.VMEM((64, Q), jnp.float32),
            pltpu.VMEM((640,), jnp.float32),
            pltpu.VMEM_SHARED((NP, Q), jnp.float32),
            pltpu.VMEM_SHARED((NP,), jnp.float32),
            pltpu.SemaphoreType.DMA,
            pltpu.SemaphoreType.DMA,
            pltpu.SemaphoreType.DMA,
            pltpu.SemaphoreType.DMA,
            pltpu.SemaphoreType.DMA,
            pltpu.SemaphoreType.DMA,
        ],
    )
    return f(xq, src, dst, p)


def kernel(nodes, edge_index, Wpre, Wlin, Wl, Wr, att, b, Wc, bc):
    loop = jnp.arange(NN, dtype=edge_index.dtype)
    pad = jnp.zeros((EP - ET,), dtype=edge_index.dtype)
    src = jnp.concatenate([edge_index[0], loop, pad])
    dst = jnp.concatenate([edge_index[1], loop, pad])

    xlh3, xrh3, xlq3 = _head(nodes, Wpre, Wlin, Wl[0], Wr[0])
    x = out = None
    for l in range(4):
        xlh = xlh3.reshape(2 * NN, HF)
        xrh = xrh3.reshape(2 * NN, HF)
        xlq = xlq3.reshape(4 * NN, Q)
        p = _score(xlh, xrh, att[l], src, dst)
        num, den = _agg(xlq, src, dst, p)
        num4 = num.reshape(4, NP, Q)
        nq = tuple(num4[q] for q in range(4))
        den2 = den.reshape(NP, 1)
        bl = b[l].reshape(1, H)
        if l < 3:
            xlh3, xrh3, xlq3 = _combine(nq, den2, bl, Wl[l + 1], Wr[l + 1])
        else:
            x, out = _final(nq, den2, bl, Wc, bc.reshape(1, Wc.shape[1]))
    return (x, out)
